# baseline (speedup 1.0000x reference)
import jax
import jax.numpy as jnp
from jax.experimental import pallas as pl
from jax.experimental.pallas import tpu as pltpu


def _sigmoid(z):
    return 0.5 * jnp.tanh(0.5 * z) + 0.5


def _cell_kernel(x_ref, m_ref, nha_ref, nhb_ref, nca_ref, ncb_ref,
                 Wiou_ref, biou_ref, Wfin_ref, bfin_ref,
                 Wf_ref, bf_ref, Waggr_ref, baggr_ref,
                 h_ref, c_ref):
    x = x_ref[...]
    m = m_ref[...]
    B, NCH2, HS = nha_ref.shape

    f_in = (jnp.dot(x, Wfin_ref[...], preferred_element_type=jnp.float32)
            + bfin_ref[...][None, :]) * m

    def half(nh, nc):
        fg = jnp.dot(nh.reshape(B * NCH2, HS), Wf_ref[...],
                     preferred_element_type=jnp.float32) + bf_ref[...][None, :]
        f = _sigmoid(fg.reshape(B, NCH2, HS) + f_in[:, None, :])
        return jnp.sum(f * nc, axis=1), jnp.sum(nh, axis=1)

    ca_a, hs_a = half(nha_ref[...], nca_ref[...])
    ca_b, hs_b = half(nhb_ref[...], ncb_ref[...])
    c_aggr = ca_a + ca_b
    h_sum = hs_a + hs_b

    iou = ((jnp.dot(x, Wiou_ref[...], preferred_element_type=jnp.float32)
            + biou_ref[...][None, :]) * m
           + jnp.dot(h_sum, Waggr_ref[...], preferred_element_type=jnp.float32)
           + baggr_ref[...][None, :])
    i = _sigmoid(iou[:, :HS])
    o = _sigmoid(iou[:, HS:2 * HS])
    u = jnp.tanh(iou[:, 2 * HS:])

    c = i * u + c_aggr
    h_ref[...] = o * jnp.tanh(c)
    c_ref[...] = c


def kernel(x_embs, x_mask, neighbour_h, neighbour_c,
           W_iou, b_iou, W_fin, b_fin, W_f, b_f, W_aggr, b_aggr,
           interpret=False):
    n, n_ch, hs = neighbour_h.shape
    xs = x_embs.shape[1]
    B = 600
    grid = ((n + B - 1) // B,)
    nch2 = n_ch // 2

    m2 = x_mask.reshape(n, 1)

    rep2 = lambda s: pl.BlockSpec(s, lambda i: (0, 0))
    rep1 = lambda s: pl.BlockSpec(s, lambda i: (0,))
    half_a = pl.BlockSpec((B, nch2, hs), lambda i: (i, 0, 0))
    half_b = pl.BlockSpec((B, nch2, hs), lambda i: (i, 1, 0))

    h, c = pl.pallas_call(
        _cell_kernel,
        grid=grid,
        in_specs=[
            pl.BlockSpec((B, xs), lambda i: (i, 0)),
            pl.BlockSpec((B, 1), lambda i: (i, 0)),
            half_a, half_b,
            pl.BlockSpec((B, nch2, hs), lambda i: (i, 0, 0)),
            pl.BlockSpec((B, nch2, hs), lambda i: (i, 1, 0)),
            rep2((xs, 3 * hs)), rep1((3 * hs,)),
            rep2((xs, hs)), rep1((hs,)),
            rep2((hs, hs)), rep1((hs,)),
            rep2((hs, 3 * hs)), rep1((3 * hs,)),
        ],
        out_specs=[
            pl.BlockSpec((B, hs), lambda i: (i, 0)),
            pl.BlockSpec((B, hs), lambda i: (i, 0)),
        ],
        out_shape=[
            jax.ShapeDtypeStruct((n, hs), jnp.float32),
            jax.ShapeDtypeStruct((n, hs), jnp.float32),
        ],
        compiler_params=pltpu.CompilerParams(
            dimension_semantics=("parallel",),
        ),
        interpret=interpret,
    )(x_embs, m2, neighbour_h, neighbour_h, neighbour_c, neighbour_c,
      W_iou, b_iou, W_fin, b_fin, W_f, b_f, W_aggr, b_aggr)
    return h, c


# speedup vs baseline: 1.0262x; 1.0262x over previous
import jax
import jax.numpy as jnp
from jax.experimental import pallas as pl
from jax.experimental.pallas import tpu as pltpu


def _sigmoid(z):
    return 0.5 * jnp.tanh(0.5 * z) + 0.5


def _cell_kernel(x_ref, m_ref, nh_ref, nc_ref,
                 Wiou_ref, biou_ref, Wfin_ref, bfin_ref,
                 Wf_ref, bf_ref, Waggr_ref, baggr_ref,
                 h_ref, c_ref):
    x = x_ref[...]
    m = m_ref[...]
    nh = nh_ref[...]
    nc = nc_ref[...]
    B, NCH, HS = nh.shape

    f_in = (jnp.dot(x, Wfin_ref[...], preferred_element_type=jnp.float32)
            + bfin_ref[...][None, :]) * m
    fg = jnp.dot(nh.reshape(B * NCH, HS), Wf_ref[...],
                 preferred_element_type=jnp.float32) + bf_ref[...][None, :]
    f = _sigmoid(fg.reshape(B, NCH, HS) + f_in[:, None, :])
    c_aggr = jnp.sum(f * nc, axis=1)

    h_sum = jnp.sum(nh, axis=1)
    iou = ((jnp.dot(x, Wiou_ref[...], preferred_element_type=jnp.float32)
            + biou_ref[...][None, :]) * m
           + jnp.dot(h_sum, Waggr_ref[...], preferred_element_type=jnp.float32)
           + baggr_ref[...][None, :])
    i = _sigmoid(iou[:, :HS])
    o = _sigmoid(iou[:, HS:2 * HS])
    u = jnp.tanh(iou[:, 2 * HS:])

    c = i * u + c_aggr
    h_ref[...] = o * jnp.tanh(c)
    c_ref[...] = c


def kernel(x_embs, x_mask, neighbour_h, neighbour_c,
           W_iou, b_iou, W_fin, b_fin, W_f, b_f, W_aggr, b_aggr,
           interpret=False):
    n, n_ch, hs = neighbour_h.shape
    xs = x_embs.shape[1]
    B = 624
    grid = ((n + B - 1) // B,)

    m2 = x_mask.reshape(n, 1)

    rep2 = lambda s: pl.BlockSpec(s, lambda i: (0, 0))
    rep1 = lambda s: pl.BlockSpec(s, lambda i: (0,))

    h, c = pl.pallas_call(
        _cell_kernel,
        grid=grid,
        in_specs=[
            pl.BlockSpec((B, xs), lambda i: (i, 0)),
            pl.BlockSpec((B, 1), lambda i: (i, 0)),
            pl.BlockSpec((B, n_ch, hs), lambda i: (i, 0, 0)),
            pl.BlockSpec((B, n_ch, hs), lambda i: (i, 0, 0)),
            rep2((xs, 3 * hs)), rep1((3 * hs,)),
            rep2((xs, hs)), rep1((hs,)),
            rep2((hs, hs)), rep1((hs,)),
            rep2((hs, 3 * hs)), rep1((3 * hs,)),
        ],
        out_specs=[
            pl.BlockSpec((B, hs), lambda i: (i, 0)),
            pl.BlockSpec((B, hs), lambda i: (i, 0)),
        ],
        out_shape=[
            jax.ShapeDtypeStruct((n, hs), jnp.float32),
            jax.ShapeDtypeStruct((n, hs), jnp.float32),
        ],
        compiler_params=pltpu.CompilerParams(
            dimension_semantics=("parallel",),
        ),
        interpret=interpret,
    )(x_embs, m2, neighbour_h, neighbour_c,
      W_iou, b_iou, W_fin, b_fin, W_f, b_f, W_aggr, b_aggr)
    return h, c


# speedup vs baseline: 1.0641x; 1.0369x over previous
import jax
import jax.numpy as jnp
from jax.experimental import pallas as pl
from jax.experimental.pallas import tpu as pltpu


def _sigmoid(z):
    return 0.5 * jnp.tanh(0.5 * z) + 0.5


def _cell_kernel(x_ref, m_ref, nh_ref, nc_ref,
                 Wiou_ref, biou_ref, Wfin_ref, bfin_ref,
                 Wf_ref, bf_ref, Waggr_ref, baggr_ref,
                 h_ref, c_ref):
    x = x_ref[...]
    m = m_ref[...]
    nh = nh_ref[...]
    nc = nc_ref[...]
    B, NCH, HS = nh.shape

    f_in = (jnp.dot(x, Wfin_ref[...], preferred_element_type=jnp.float32)
            + bfin_ref[...][None, :]) * m
    fg = jnp.dot(nh.reshape(B * NCH, HS), Wf_ref[...],
                 preferred_element_type=jnp.float32) + bf_ref[...][None, :]
    f = _sigmoid(fg.reshape(B, NCH, HS) + f_in[:, None, :])
    c_aggr = jnp.sum(f * nc, axis=1)

    h_sum = jnp.sum(nh, axis=1)
    iou = ((jnp.dot(x, Wiou_ref[...], preferred_element_type=jnp.float32)
            + biou_ref[...][None, :]) * m
           + jnp.dot(h_sum, Waggr_ref[...], preferred_element_type=jnp.float32)
           + baggr_ref[...][None, :])
    i = _sigmoid(iou[:, :HS])
    o = _sigmoid(iou[:, HS:2 * HS])
    u = jnp.tanh(iou[:, 2 * HS:])

    c = i * u + c_aggr
    h_ref[...] = o * jnp.tanh(c)
    c_ref[...] = c


def kernel(x_embs, x_mask, neighbour_h, neighbour_c,
           W_iou, b_iou, W_fin, b_fin, W_f, b_f, W_aggr, b_aggr,
           interpret=False):
    n, n_ch, hs = neighbour_h.shape
    xs = x_embs.shape[1]
    B = 592
    grid = ((n + B - 1) // B,)

    m2 = x_mask.reshape(n, 1)

    rep2 = lambda s: pl.BlockSpec(s, lambda i: (0, 0))
    rep1 = lambda s: pl.BlockSpec(s, lambda i: (0,))

    h, c = pl.pallas_call(
        _cell_kernel,
        grid=grid,
        in_specs=[
            pl.BlockSpec((B, xs), lambda i: (i, 0)),
            pl.BlockSpec((B, 1), lambda i: (i, 0)),
            pl.BlockSpec((B, n_ch, hs), lambda i: (i, 0, 0)),
            pl.BlockSpec((B, n_ch, hs), lambda i: (i, 0, 0)),
            rep2((xs, 3 * hs)), rep1((3 * hs,)),
            rep2((xs, hs)), rep1((hs,)),
            rep2((hs, hs)), rep1((hs,)),
            rep2((hs, 3 * hs)), rep1((3 * hs,)),
        ],
        out_specs=[
            pl.BlockSpec((B, hs), lambda i: (i, 0)),
            pl.BlockSpec((B, hs), lambda i: (i, 0)),
        ],
        out_shape=[
            jax.ShapeDtypeStruct((n, hs), jnp.float32),
            jax.ShapeDtypeStruct((n, hs), jnp.float32),
        ],
        compiler_params=pltpu.CompilerParams(
            dimension_semantics=("parallel",),
        ),
        interpret=interpret,
    )(x_embs, m2, neighbour_h, neighbour_c,
      W_iou, b_iou, W_fin, b_fin, W_f, b_f, W_aggr, b_aggr)
    return h, c
